# initial kernel scaffold (unmeasured)
import jax
import jax.numpy as jnp
from jax import lax
from jax.experimental import pallas as pl
from jax.experimental.pallas import tpu as pltpu

N_DEV = 8


def kernel(x, w_mat):
    m_per, k = x.shape
    _, n_per = w_mat.shape
    m_tot = N_DEV * m_per

    def body(x_ref, w_ref, out_ref, comm_ref, amax_ref,
             send_sems, recv_sems, amax_send_sems, amax_recv_sems,
             credit_sem):
        my = lax.axis_index("i")
        left = lax.rem(my + (N_DEV - 1), N_DEV)
        right = lax.rem(my + 1, N_DEV)

        def hop_rdma(h):
            s = h % 2
            r = (h + 1) % 2
            return pltpu.make_async_remote_copy(
                src_ref=comm_ref.at[s],
                dst_ref=comm_ref.at[r],
                send_sem=send_sems.at[s],
                recv_sem=recv_sems.at[r],
                device_id=(right,),
                device_id_type=pl.DeviceIdType.MESH,
            )

        comm_ref[0, :, :] = x_ref[:, :]
        rdma = hop_rdma(0)
        rdma.start()

        out_ref[pl.ds(my * m_per, m_per), :] = jnp.dot(
            x_ref[...], w_ref[...], preferred_element_type=jnp.float32
        )

        for h in range(N_DEV - 1):
            r = (h + 1) % 2
            rdma.wait()
            if h < N_DEV - 2:
                pl.semaphore_signal(
                    credit_sem, inc=1,
                    device_id=(left,), device_id_type=pl.DeviceIdType.MESH,
                )
                pl.semaphore_wait(credit_sem, 1)
                rdma = hop_rdma(h + 1)
                rdma.start()
            origin = lax.rem(my + (N_DEV - 1 - h), N_DEV)
            out_ref[pl.ds(origin * m_per, m_per), :] = jnp.dot(
                comm_ref[r], w_ref[...], preferred_element_type=jnp.float32
            )

        local_amax = jnp.maximum(jnp.max(out_ref[...]), 0.0)
        amax_ref[pl.ds(my, 1), :] = jnp.full((1, 128), local_amax, jnp.float32)
        sends = []
        for j in range(1, N_DEV):
            tgt = lax.rem(my + j, N_DEV)
            a = pltpu.make_async_remote_copy(
                src_ref=amax_ref.at[pl.ds(my, 1)],
                dst_ref=amax_ref.at[pl.ds(my, 1)],
                send_sem=amax_send_sems.at[tgt],
                recv_sem=amax_recv_sems.at[my],
                device_id=(tgt,),
                device_id_type=pl.DeviceIdType.MESH,
            )
            a.start()
            sends.append(a)
        for a in sends:
            a.wait_send()
        for j in range(1, N_DEV):
            src = lax.rem(my + j, N_DEV)
            rcv = pltpu.make_async_remote_copy(
                src_ref=amax_ref.at[pl.ds(src, 1)],
                dst_ref=amax_ref.at[pl.ds(src, 1)],
                send_sem=amax_send_sems.at[src],
                recv_sem=amax_recv_sems.at[src],
                device_id=(src,),
                device_id_type=pl.DeviceIdType.MESH,
            )
            rcv.wait_recv()

        amax = jnp.max(amax_ref[...])
        scale = amax / 448.0
        y = jnp.maximum(out_ref[...], 0.0)
        q = jnp.minimum(y / scale, 448.0)
        q = q.astype(jnp.float8_e4m3fn).astype(jnp.float32)
        out_ref[...] = q * scale

    return pl.pallas_call(
        body,
        out_shape=jax.ShapeDtypeStruct((m_tot, n_per), jnp.float32),
        in_specs=[
            pl.BlockSpec(memory_space=pltpu.VMEM),
            pl.BlockSpec(memory_space=pltpu.VMEM),
        ],
        out_specs=pl.BlockSpec(memory_space=pltpu.VMEM),
        scratch_shapes=[
            pltpu.VMEM((2, m_per, k), jnp.float32),
            pltpu.VMEM((N_DEV, 128), jnp.float32),
            pltpu.SemaphoreType.DMA((2,)),
            pltpu.SemaphoreType.DMA((2,)),
            pltpu.SemaphoreType.DMA((N_DEV,)),
            pltpu.SemaphoreType.DMA((N_DEV,)),
            pltpu.SemaphoreType.REGULAR,
        ],
        compiler_params=pltpu.CompilerParams(collective_id=0),
    )(x, w_mat)


# baseline (device time: 672295 ns/iter reference)
import jax
import jax.numpy as jnp
from jax import lax
from jax.experimental import pallas as pl
from jax.experimental.pallas import tpu as pltpu

N_DEV = 8


def kernel(x, w_mat):
    m_per, k = x.shape
    _, n_per = w_mat.shape
    m_tot = N_DEV * m_per

    def body(x_ref, w_ref, out_ref, comm_ref, amax_ref,
             send_sems, recv_sems, amax_send_sems, amax_recv_sems,
             credit_sem):
        my = lax.axis_index("i")
        left = lax.rem(my + (N_DEV - 1), N_DEV)
        right = lax.rem(my + 1, N_DEV)

        def hop_rdma(h):
            s = h % 2
            r = (h + 1) % 2
            return pltpu.make_async_remote_copy(
                src_ref=comm_ref.at[s],
                dst_ref=comm_ref.at[r],
                send_sem=send_sems.at[s],
                recv_sem=recv_sems.at[r],
                device_id=(right,),
                device_id_type=pl.DeviceIdType.MESH,
            )

        comm_ref[0, :, :] = x_ref[:, :]
        rdma = hop_rdma(0)
        rdma.start()

        out_ref[pl.ds(my * m_per, m_per), :] = jnp.dot(
            x_ref[...], w_ref[...], preferred_element_type=jnp.float32
        )

        for h in range(N_DEV - 1):
            r = (h + 1) % 2
            rdma.wait()
            if h < N_DEV - 2:
                pl.semaphore_signal(
                    credit_sem, inc=1,
                    device_id=(left,), device_id_type=pl.DeviceIdType.MESH,
                )
                pl.semaphore_wait(credit_sem, 1)
                rdma = hop_rdma(h + 1)
                rdma.start()
            origin = lax.rem(my + (N_DEV - 1 - h), N_DEV)
            out_ref[pl.ds(origin * m_per, m_per), :] = jnp.dot(
                comm_ref[r], w_ref[...], preferred_element_type=jnp.float32
            )

        local_amax = jnp.maximum(jnp.max(out_ref[...]), 0.0)
        amax_ref[pl.ds(my, 1), :] = jnp.full((1, 128), local_amax, jnp.float32)
        sends = []
        for j in range(1, N_DEV):
            tgt = lax.rem(my + j, N_DEV)
            a = pltpu.make_async_remote_copy(
                src_ref=amax_ref.at[pl.ds(my, 1)],
                dst_ref=amax_ref.at[pl.ds(my, 1)],
                send_sem=amax_send_sems.at[tgt],
                recv_sem=amax_recv_sems.at[my],
                device_id=(tgt,),
                device_id_type=pl.DeviceIdType.MESH,
            )
            a.start()
            sends.append(a)
        for a in sends:
            a.wait_send()
        for j in range(1, N_DEV):
            src = lax.rem(my + j, N_DEV)
            rcv = pltpu.make_async_remote_copy(
                src_ref=amax_ref.at[pl.ds(src, 1)],
                dst_ref=amax_ref.at[pl.ds(src, 1)],
                send_sem=amax_send_sems.at[src],
                recv_sem=amax_recv_sems.at[src],
                device_id=(src,),
                device_id_type=pl.DeviceIdType.MESH,
            )
            rcv.wait_recv()

        amax = jnp.max(amax_ref[...])
        scale = amax / 448.0
        y = jnp.maximum(out_ref[...], 0.0)
        q = jnp.minimum(y / scale, 448.0)
        q = q.astype(jnp.float8_e4m3fn).astype(jnp.float32)
        out_ref[...] = q * scale

    return pl.pallas_call(
        body,
        out_shape=jax.ShapeDtypeStruct((m_tot, n_per), jnp.float32),
        in_specs=[
            pl.BlockSpec(memory_space=pltpu.VMEM),
            pl.BlockSpec(memory_space=pltpu.VMEM),
        ],
        out_specs=pl.BlockSpec(memory_space=pltpu.VMEM),
        scratch_shapes=[
            pltpu.VMEM((2, m_per, k), jnp.float32),
            pltpu.VMEM((N_DEV, 128), jnp.float32),
            pltpu.SemaphoreType.DMA((2,)),
            pltpu.SemaphoreType.DMA((2,)),
            pltpu.SemaphoreType.DMA((N_DEV,)),
            pltpu.SemaphoreType.DMA((N_DEV,)),
            pltpu.SemaphoreType.REGULAR,
        ],
    )(x, w_mat)


# device time: 364228 ns/iter; 1.8458x vs baseline; 1.8458x over previous
import jax
import jax.numpy as jnp
from jax import lax
from jax.experimental import pallas as pl
from jax.experimental.pallas import tpu as pltpu

N_DEV = 8


def kernel(x, w_mat):
    m_per, k = x.shape
    _, n_per = w_mat.shape
    m_tot = N_DEV * m_per
    m_half = m_per // 2

    def body(x_ref, w_ref, out_ref, cw_ref, ccw_ref, amax_ref,
             cw_send_sems, cw_recv_sems, ccw_send_sems, ccw_recv_sems,
             amax_send_sems, amax_recv_sems, cw_credit, ccw_credit):
        my = lax.axis_index("i")
        left = lax.rem(my + (N_DEV - 1), N_DEV)
        right = lax.rem(my + 1, N_DEV)

        def dot(a, b):
            return jnp.dot(a, b, preferred_element_type=jnp.float32,
                           precision=lax.Precision.HIGHEST)

        def hop_rdma(h, comm_ref, send_sems, recv_sems, tgt):
            s = h % 2
            r = (h + 1) % 2
            return pltpu.make_async_remote_copy(
                src_ref=comm_ref.at[s],
                dst_ref=comm_ref.at[r],
                send_sem=send_sems.at[s],
                recv_sem=recv_sems.at[r],
                device_id=(tgt,),
                device_id_type=pl.DeviceIdType.MESH,
            )

        cw_ref[0, :, :] = x_ref[:m_half, :]
        ccw_ref[0, :, :] = x_ref[m_half:, :]
        cw = hop_rdma(0, cw_ref, cw_send_sems, cw_recv_sems, right)
        ccw = hop_rdma(0, ccw_ref, ccw_send_sems, ccw_recv_sems, left)
        cw.start()
        ccw.start()

        out_ref[pl.ds(my * m_per, m_per), :] = dot(x_ref[...], w_ref[...])

        for h in range(N_DEV - 1):
            r = (h + 1) % 2
            cw.wait()
            ccw.wait()
            if h < N_DEV - 2:
                pl.semaphore_signal(
                    cw_credit, inc=1,
                    device_id=(left,), device_id_type=pl.DeviceIdType.MESH,
                )
                pl.semaphore_signal(
                    ccw_credit, inc=1,
                    device_id=(right,), device_id_type=pl.DeviceIdType.MESH,
                )
                pl.semaphore_wait(cw_credit, 1)
                pl.semaphore_wait(ccw_credit, 1)
                cw = hop_rdma(h + 1, cw_ref, cw_send_sems, cw_recv_sems, right)
                ccw = hop_rdma(h + 1, ccw_ref, ccw_send_sems, ccw_recv_sems, left)
                cw.start()
                ccw.start()
            o_cw = lax.rem(my + (N_DEV - 1 - h), N_DEV)
            o_ccw = lax.rem(my + h + 1, N_DEV)
            out_ref[pl.ds(o_cw * m_per, m_half), :] = dot(cw_ref[r], w_ref[...])
            out_ref[pl.ds(o_ccw * m_per + m_half, m_half), :] = dot(
                ccw_ref[r], w_ref[...]
            )

        local_amax = jnp.maximum(jnp.max(out_ref[...]), 0.0)
        amax_ref[pl.ds(my, 1), :] = jnp.full((1, 128), local_amax, jnp.float32)
        sends = []
        for j in range(1, N_DEV):
            tgt = lax.rem(my + j, N_DEV)
            a = pltpu.make_async_remote_copy(
                src_ref=amax_ref.at[pl.ds(my, 1)],
                dst_ref=amax_ref.at[pl.ds(my, 1)],
                send_sem=amax_send_sems.at[tgt],
                recv_sem=amax_recv_sems.at[my],
                device_id=(tgt,),
                device_id_type=pl.DeviceIdType.MESH,
            )
            a.start()
            sends.append(a)
        for a in sends:
            a.wait_send()
        for j in range(1, N_DEV):
            src = lax.rem(my + j, N_DEV)
            rcv = pltpu.make_async_remote_copy(
                src_ref=amax_ref.at[pl.ds(src, 1)],
                dst_ref=amax_ref.at[pl.ds(src, 1)],
                send_sem=amax_send_sems.at[src],
                recv_sem=amax_recv_sems.at[src],
                device_id=(src,),
                device_id_type=pl.DeviceIdType.MESH,
            )
            rcv.wait_recv()

        amax = jnp.max(amax_ref[...])
        scale = amax / 448.0
        y = jnp.maximum(out_ref[...], 0.0)
        q = jnp.minimum(y / scale, 448.0)
        q = q.astype(jnp.float8_e4m3fn).astype(jnp.float32)
        out_ref[...] = q * scale

    return pl.pallas_call(
        body,
        out_shape=jax.ShapeDtypeStruct((m_tot, n_per), jnp.float32),
        in_specs=[
            pl.BlockSpec(memory_space=pltpu.VMEM),
            pl.BlockSpec(memory_space=pltpu.VMEM),
        ],
        out_specs=pl.BlockSpec(memory_space=pltpu.VMEM),
        scratch_shapes=[
            pltpu.VMEM((2, m_half, k), jnp.float32),
            pltpu.VMEM((2, m_half, k), jnp.float32),
            pltpu.VMEM((N_DEV, 128), jnp.float32),
            pltpu.SemaphoreType.DMA((2,)),
            pltpu.SemaphoreType.DMA((2,)),
            pltpu.SemaphoreType.DMA((2,)),
            pltpu.SemaphoreType.DMA((2,)),
            pltpu.SemaphoreType.DMA((N_DEV,)),
            pltpu.SemaphoreType.DMA((N_DEV,)),
            pltpu.SemaphoreType.REGULAR,
            pltpu.SemaphoreType.REGULAR,
        ],
    )(x, w_mat)


# device time: 279045 ns/iter; 2.4093x vs baseline; 1.3053x over previous
import jax
import jax.numpy as jnp
from jax import lax
from jax.experimental import pallas as pl
from jax.experimental.pallas import tpu as pltpu

N_DEV = 8
PART_ROWS = (176, 168, 168)
PART_OFFS = (0, 176, 344)


def kernel(x, w_mat):
    m_per, k = x.shape
    _, n_per = w_mat.shape
    m_tot = N_DEV * m_per

    def body(x_ref, w_ref, out_ref, p0, p1, p2, stage_ref, amax_ref,
             send_sems, recv_sems, amax_send_sems, amax_recv_sems,
             copy_sems, stage_sems):
        parts = (p0, p1, p2)
        my = lax.axis_index("i")

        zc = my // 4
        p = my - 4 * zc
        yc = p // 2
        xc = jnp.bitwise_xor(p - 2 * yc, yc)

        def pos(x_, y_, z_):
            return 4 * z_ + 2 * y_ + jnp.bitwise_xor(x_, y_)

        nbr = (pos(1 - xc, yc, zc), pos(xc, 1 - yc, zc), pos(xc, yc, 1 - zc))

        def origin(j, s):
            c = [xc, yc, zc]
            for t in range(3):
                if (s >> t) & 1:
                    e = (j + t) % 3
                    c[e] = 1 - c[e]
            return pos(*c)

        def dot(a):
            return jnp.dot(a, w_ref[...], preferred_element_type=jnp.float32,
                           precision=lax.Precision.HIGHEST)

        toggle = [0]
        queue = []

        def stage_unit(src, rows, out_off):
            s = toggle[0]
            toggle[0] ^= 1
            cp = pltpu.make_async_copy(
                src, stage_ref.at[s, pl.ds(0, rows)], stage_sems.at[s]
            )
            cp.start()
            queue.append((cp, s, rows, out_off))

        def finish_unit():
            cp, s, rows, out_off = queue.pop(0)
            cp.wait()
            out_ref[pl.ds(out_off, rows), :] = dot(stage_ref[s, pl.ds(0, rows)])

        def push(src, rows, out_off):
            if len(queue) == 2:
                finish_unit()
            stage_unit(src, rows, out_off)

        def exchange(j, r, src_slot, dst_slot, sem_idx, src=None):
            return pltpu.make_async_remote_copy(
                src_ref=parts[j].at[src_slot] if src is None else src,
                dst_ref=parts[j].at[dst_slot],
                send_sem=send_sems.at[j, sem_idx],
                recv_sem=recv_sems.at[j, sem_idx],
                device_id=(nbr[(j + r) % 3],),
                device_id_type=pl.DeviceIdType.MESH,
            )

        r0 = []
        for j in range(3):
            rd = exchange(j, 0, src_slot=0, dst_slot=1, sem_idx=0,
                          src=x_ref.at[pl.ds(PART_OFFS[j], PART_ROWS[j])])
            rd.start()
            r0.append(rd)
        copies = []
        for j in range(3):
            cp = pltpu.make_async_copy(
                x_ref.at[pl.ds(PART_OFFS[j], PART_ROWS[j])],
                parts[j].at[0],
                copy_sems.at[j],
            )
            cp.start()
            copies.append(cp)

        push(x_ref, m_per, my * m_per)

        for cp in copies:
            cp.wait()
        for rd in r0:
            rd.wait()

        r1 = []
        for j in range(3):
            for i, (s_src, s_dst) in enumerate(((0, 2), (1, 3))):
                rd = exchange(j, 1, s_src, s_dst, sem_idx=1 + i)
                rd.start()
                r1.append(rd)
        for j in range(3):
            push(parts[j].at[1], PART_ROWS[j],
                 origin(j, 1) * m_per + PART_OFFS[j])
        for rd in r1:
            rd.wait()

        r2a, r2b = [], []
        for j in range(3):
            for i, (s_src, s_dst) in enumerate(((0, 4), (1, 5))):
                rd = exchange(j, 2, s_src, s_dst, sem_idx=3 + i)
                rd.start()
                r2a.append(rd)
        for j in range(3):
            for i, (s_src, s_dst) in enumerate(((2, 6), (3, 7))):
                rd = exchange(j, 2, s_src, s_dst, sem_idx=5 + i)
                rd.start()
                r2b.append(rd)
        for j in range(3):
            for s in (2, 3):
                push(parts[j].at[s], PART_ROWS[j],
                     origin(j, s) * m_per + PART_OFFS[j])
        for rd in r2a:
            rd.wait()
        for j in range(3):
            for s in (4, 5):
                push(parts[j].at[s], PART_ROWS[j],
                     origin(j, s) * m_per + PART_OFFS[j])
        for rd in r2b:
            rd.wait()
        for j in range(3):
            for s in (6, 7):
                push(parts[j].at[s], PART_ROWS[j],
                     origin(j, s) * m_per + PART_OFFS[j])
        while queue:
            finish_unit()

        local_amax = jnp.maximum(jnp.max(out_ref[...]), 0.0)
        amax_ref[pl.ds(my, 1), :] = jnp.full((1, 128), local_amax, jnp.float32)
        sends = []
        for j in range(1, N_DEV):
            tgt = lax.rem(my + j, N_DEV)
            a = pltpu.make_async_remote_copy(
                src_ref=amax_ref.at[pl.ds(my, 1)],
                dst_ref=amax_ref.at[pl.ds(my, 1)],
                send_sem=amax_send_sems.at[tgt],
                recv_sem=amax_recv_sems.at[my],
                device_id=(tgt,),
                device_id_type=pl.DeviceIdType.MESH,
            )
            a.start()
            sends.append(a)
        for a in sends:
            a.wait_send()
        for j in range(1, N_DEV):
            src = lax.rem(my + j, N_DEV)
            rcv = pltpu.make_async_remote_copy(
                src_ref=amax_ref.at[pl.ds(src, 1)],
                dst_ref=amax_ref.at[pl.ds(src, 1)],
                send_sem=amax_send_sems.at[src],
                recv_sem=amax_recv_sems.at[src],
                device_id=(src,),
                device_id_type=pl.DeviceIdType.MESH,
            )
            rcv.wait_recv()

        amax = jnp.max(amax_ref[...])
        scale = amax / 448.0
        y = jnp.maximum(out_ref[...], 0.0)
        q = jnp.minimum(y / scale, 448.0)
        q = q.astype(jnp.float8_e4m3fn).astype(jnp.float32)
        out_ref[...] = q * scale

    out, _, _, _ = pl.pallas_call(
        body,
        out_shape=(
            jax.ShapeDtypeStruct((m_tot, n_per), jnp.float32),
            jax.ShapeDtypeStruct((N_DEV, PART_ROWS[0], k), jnp.float32),
            jax.ShapeDtypeStruct((N_DEV, PART_ROWS[1], k), jnp.float32),
            jax.ShapeDtypeStruct((N_DEV, PART_ROWS[2], k), jnp.float32),
        ),
        in_specs=[
            pl.BlockSpec(memory_space=pl.ANY),
            pl.BlockSpec(memory_space=pltpu.VMEM),
        ],
        out_specs=(
            pl.BlockSpec(memory_space=pltpu.VMEM),
            pl.BlockSpec(memory_space=pl.ANY),
            pl.BlockSpec(memory_space=pl.ANY),
            pl.BlockSpec(memory_space=pl.ANY),
        ),
        scratch_shapes=[
            pltpu.VMEM((2, m_per, k), jnp.float32),
            pltpu.VMEM((N_DEV, 128), jnp.float32),
            pltpu.SemaphoreType.DMA((3, 7)),
            pltpu.SemaphoreType.DMA((3, 7)),
            pltpu.SemaphoreType.DMA((N_DEV,)),
            pltpu.SemaphoreType.DMA((N_DEV,)),
            pltpu.SemaphoreType.DMA((3,)),
            pltpu.SemaphoreType.DMA((2,)),
        ],
    )(x, w_mat)
    return out


# device time: 252299 ns/iter; 2.6647x vs baseline; 1.1060x over previous
import jax
import jax.numpy as jnp
from jax import lax
from jax.experimental import pallas as pl
from jax.experimental.pallas import tpu as pltpu

N_DEV = 8
PART_ROWS = (176, 168, 168)
PART_OFFS = (0, 176, 344)


def kernel(x, w_mat):
    m_per, k = x.shape
    _, n_per = w_mat.shape
    m_tot = N_DEV * m_per

    def body(x_ref, w_ref, out_ref, p0, p1, p2, land0, land1, land2,
             stage_ref, amax_ref, send_sems, recv_sems,
             amax_send_sems, amax_recv_sems, stage_sems):
        parts = (p0, p1, p2)
        landing = (land0, land1, land2)
        my = lax.axis_index("i")

        zc = my // 4
        p = my - 4 * zc
        yc = p // 2
        xc = jnp.bitwise_xor(p - 2 * yc, yc)

        def pos(x_, y_, z_):
            return 4 * z_ + 2 * y_ + jnp.bitwise_xor(x_, y_)

        nbr = (pos(1 - xc, yc, zc), pos(xc, 1 - yc, zc), pos(xc, yc, 1 - zc))

        def origin(j, s):
            c = [xc, yc, zc]
            for t in range(3):
                if (s >> t) & 1:
                    e = (j + t) % 3
                    c[e] = 1 - c[e]
            return pos(*c)

        def xslice(j):
            return x_ref.at[pl.ds(PART_OFFS[j], PART_ROWS[j])]

        running_amax = [jnp.float32(0.0)]

        def emit(j, yv, slot):
            running_amax[0] = jnp.maximum(running_amax[0], jnp.max(yv))
            out_ref[pl.ds(origin(j, slot) * m_per + PART_OFFS[j],
                          PART_ROWS[j]), :] = yv

        def dot(a):
            return jnp.dot(a, w_ref[...], preferred_element_type=jnp.float32,
                           precision=lax.Precision.HIGHEST)

        toggle = [0]
        queue = []

        def push(src, j, slot):
            if len(queue) == 2:
                finish_unit()
            s = toggle[0]
            toggle[0] ^= 1
            rows = PART_ROWS[j]
            cp = pltpu.make_async_copy(
                src, stage_ref.at[s, pl.ds(0, rows)], stage_sems.at[s]
            )
            cp.start()
            queue.append((cp, s, j, slot))

        def finish_unit():
            cp, s, j, slot = queue.pop(0)
            cp.wait()
            emit(j, dot(stage_ref[s, pl.ds(0, PART_ROWS[j])]), slot)

        def exchange(j, r, src, dst, sem_idx):
            return pltpu.make_async_remote_copy(
                src_ref=src,
                dst_ref=dst,
                send_sem=send_sems.at[j, sem_idx],
                recv_sem=recv_sems.at[j, sem_idx],
                device_id=(nbr[(j + r) % 3],),
                device_id_type=pl.DeviceIdType.MESH,
            )

        r0 = []
        for j in range(3):
            rd = exchange(j, 0, xslice(j), parts[j].at[1], sem_idx=0)
            rd.start()
            r0.append(rd)
        for j in range(3):
            push(xslice(j), j, 0)
        for rd in r0:
            rd.wait()

        r1a, r1b = [], []
        for j in range(3):
            rd = exchange(j, 1, xslice(j), parts[j].at[2], sem_idx=1)
            rd.start()
            r1a.append(rd)
            rd = exchange(j, 1, parts[j].at[1], parts[j].at[3], sem_idx=2)
            rd.start()
            r1b.append(rd)
        for j in range(3):
            push(parts[j].at[1], j, 1)
        for rd in r1a:
            rd.wait()
        for j in range(3):
            push(parts[j].at[2], j, 2)
        for rd in r1b:
            rd.wait()

        r2 = [[], [], [], []]
        for j in range(3):
            srcs = (xslice(j), parts[j].at[1], parts[j].at[2], parts[j].at[3])
            for i in range(4):
                rd = exchange(j, 2, srcs[i], landing[j].at[i], sem_idx=3 + i)
                rd.start()
                r2[i].append(rd)
        for j in range(3):
            push(parts[j].at[3], j, 3)
        for i in range(4):
            for rd in r2[i]:
                rd.wait()
            if i == 0:
                while queue:
                    finish_unit()
            for j in range(3):
                emit(j, dot(landing[j][i]), 4 + i)

        local_amax = jnp.maximum(running_amax[0], 0.0)
        amax_ref[pl.ds(my, 1), :] = jnp.full((1, 128), local_amax, jnp.float32)
        sends = []
        for j in range(1, N_DEV):
            tgt = lax.rem(my + j, N_DEV)
            a = pltpu.make_async_remote_copy(
                src_ref=amax_ref.at[pl.ds(my, 1)],
                dst_ref=amax_ref.at[pl.ds(my, 1)],
                send_sem=amax_send_sems.at[tgt],
                recv_sem=amax_recv_sems.at[my],
                device_id=(tgt,),
                device_id_type=pl.DeviceIdType.MESH,
            )
            a.start()
            sends.append(a)
        for a in sends:
            a.wait_send()
        for j in range(1, N_DEV):
            src = lax.rem(my + j, N_DEV)
            rcv = pltpu.make_async_remote_copy(
                src_ref=amax_ref.at[pl.ds(src, 1)],
                dst_ref=amax_ref.at[pl.ds(src, 1)],
                send_sem=amax_send_sems.at[src],
                recv_sem=amax_recv_sems.at[src],
                device_id=(src,),
                device_id_type=pl.DeviceIdType.MESH,
            )
            rcv.wait_recv()

        amax = jnp.max(amax_ref[...])
        scale = amax / 448.0
        y = jnp.maximum(out_ref[...], 0.0)
        q = jnp.minimum(y / scale, 448.0)
        q = q.astype(jnp.float8_e4m3fn).astype(jnp.float32)
        out_ref[...] = q * scale

    out, _, _, _ = pl.pallas_call(
        body,
        out_shape=(
            jax.ShapeDtypeStruct((m_tot, n_per), jnp.float32),
            jax.ShapeDtypeStruct((N_DEV, PART_ROWS[0], k), jnp.float32),
            jax.ShapeDtypeStruct((N_DEV, PART_ROWS[1], k), jnp.float32),
            jax.ShapeDtypeStruct((N_DEV, PART_ROWS[2], k), jnp.float32),
        ),
        in_specs=[
            pl.BlockSpec(memory_space=pl.ANY),
            pl.BlockSpec(memory_space=pltpu.VMEM),
        ],
        out_specs=(
            pl.BlockSpec(memory_space=pltpu.VMEM),
            pl.BlockSpec(memory_space=pl.ANY),
            pl.BlockSpec(memory_space=pl.ANY),
            pl.BlockSpec(memory_space=pl.ANY),
        ),
        scratch_shapes=[
            pltpu.VMEM((4, PART_ROWS[0], k), jnp.float32),
            pltpu.VMEM((4, PART_ROWS[1], k), jnp.float32),
            pltpu.VMEM((4, PART_ROWS[2], k), jnp.float32),
            pltpu.VMEM((2, PART_ROWS[0], k), jnp.float32),
            pltpu.VMEM((N_DEV, 128), jnp.float32),
            pltpu.SemaphoreType.DMA((3, 7)),
            pltpu.SemaphoreType.DMA((3, 7)),
            pltpu.SemaphoreType.DMA((N_DEV,)),
            pltpu.SemaphoreType.DMA((N_DEV,)),
            pltpu.SemaphoreType.DMA((2,)),
        ],
        compiler_params=pltpu.CompilerParams(
            vmem_limit_bytes=52 * 1024 * 1024,
        ),
    )(x, w_mat)
    return out


# device time: 248057 ns/iter; 2.7102x vs baseline; 1.0171x over previous
import jax
import jax.numpy as jnp
from jax import lax
from jax.experimental import pallas as pl
from jax.experimental.pallas import tpu as pltpu

N_DEV = 8
PART_ROWS = (176, 168, 168)
PART_OFFS = (0, 176, 344)


def kernel(x, w_mat):
    m_per, k = x.shape
    _, n_per = w_mat.shape
    m_tot = N_DEV * m_per

    def body(x_ref, w_ref, out_ref, p0, p1, p2, land0, land1, land2,
             stage_ref, amax_ref, send_sems, recv_sems,
             amax_send_sems, amax_recv_sems, stage_sems):
        parts = (p0, p1, p2)
        landing = (land0, land1, land2)
        my = lax.axis_index("i")

        zc = my // 4
        p = my - 4 * zc
        yc = p // 2
        xc = jnp.bitwise_xor(p - 2 * yc, yc)

        def pos(x_, y_, z_):
            return 4 * z_ + 2 * y_ + jnp.bitwise_xor(x_, y_)

        nbr = (pos(1 - xc, yc, zc), pos(xc, 1 - yc, zc), pos(xc, yc, 1 - zc))

        def origin(j, s):
            c = [xc, yc, zc]
            for t in range(3):
                if (s >> t) & 1:
                    e = (j + t) % 3
                    c[e] = 1 - c[e]
            return pos(*c)

        def xslice(j):
            return x_ref.at[pl.ds(PART_OFFS[j], PART_ROWS[j])]

        running_amax = [jnp.float32(0.0)]

        def emit(j, yv, slot):
            running_amax[0] = jnp.maximum(running_amax[0], jnp.max(yv))
            out_ref[pl.ds(origin(j, slot) * m_per + PART_OFFS[j],
                          PART_ROWS[j]), :] = yv

        def dot(a):
            return jnp.dot(a, w_ref[...], preferred_element_type=jnp.float32,
                           precision=lax.Precision.HIGHEST)

        toggle = [0]
        queue = []

        def push(src, j, slot):
            if len(queue) == 2:
                finish_unit()
            s = toggle[0]
            toggle[0] ^= 1
            rows = PART_ROWS[j]
            cp = pltpu.make_async_copy(
                src, stage_ref.at[s, pl.ds(0, rows)], stage_sems.at[s]
            )
            cp.start()
            queue.append((cp, s, j, slot))

        def finish_unit():
            cp, s, j, slot = queue.pop(0)
            cp.wait()
            emit(j, dot(stage_ref[s, pl.ds(0, PART_ROWS[j])]), slot)

        def exchange(j, r, src, dst, sem_idx):
            return pltpu.make_async_remote_copy(
                src_ref=src,
                dst_ref=dst,
                send_sem=send_sems.at[j, sem_idx],
                recv_sem=recv_sems.at[j, sem_idx],
                device_id=(nbr[(j + r) % 3],),
                device_id_type=pl.DeviceIdType.MESH,
            )

        r0, r1a, r1b = [], [], []
        r2 = [[], [], [], []]
        for j in range(3):
            rd = exchange(j, 0, xslice(j), parts[j].at[1], sem_idx=0)
            rd.start()
            r0.append(rd)
            rd = exchange(j, 1, xslice(j), parts[j].at[2], sem_idx=1)
            rd.start()
            r1a.append(rd)
            rd = exchange(j, 2, xslice(j), landing[j].at[0], sem_idx=3)
            rd.start()
            r2[0].append(rd)
        for j in range(3):
            push(xslice(j), j, 0)

        for rd in r0:
            rd.wait()
        for j in range(3):
            rd = exchange(j, 1, parts[j].at[1], parts[j].at[3], sem_idx=2)
            rd.start()
            r1b.append(rd)
            rd = exchange(j, 2, parts[j].at[1], landing[j].at[1], sem_idx=4)
            rd.start()
            r2[1].append(rd)
        for j in range(3):
            push(parts[j].at[1], j, 1)

        for rd in r1a:
            rd.wait()
        for j in range(3):
            rd = exchange(j, 2, parts[j].at[2], landing[j].at[2], sem_idx=5)
            rd.start()
            r2[2].append(rd)
        for j in range(3):
            push(parts[j].at[2], j, 2)

        for rd in r1b:
            rd.wait()
        for j in range(3):
            rd = exchange(j, 2, parts[j].at[3], landing[j].at[3], sem_idx=6)
            rd.start()
            r2[3].append(rd)
        for j in range(3):
            push(parts[j].at[3], j, 3)

        for i in range(4):
            for rd in r2[i]:
                rd.wait()
            if i == 0:
                while queue:
                    finish_unit()
            for j in range(3):
                emit(j, dot(landing[j][i]), 4 + i)

        local_amax = jnp.maximum(running_amax[0], 0.0)
        amax_ref[pl.ds(my, 1), :] = jnp.full((1, 128), local_amax, jnp.float32)
        sends = []
        for j in range(1, N_DEV):
            tgt = lax.rem(my + j, N_DEV)
            a = pltpu.make_async_remote_copy(
                src_ref=amax_ref.at[pl.ds(my, 1)],
                dst_ref=amax_ref.at[pl.ds(my, 1)],
                send_sem=amax_send_sems.at[tgt],
                recv_sem=amax_recv_sems.at[my],
                device_id=(tgt,),
                device_id_type=pl.DeviceIdType.MESH,
            )
            a.start()
            sends.append(a)
        for a in sends:
            a.wait_send()
        for j in range(1, N_DEV):
            src = lax.rem(my + j, N_DEV)
            rcv = pltpu.make_async_remote_copy(
                src_ref=amax_ref.at[pl.ds(src, 1)],
                dst_ref=amax_ref.at[pl.ds(src, 1)],
                send_sem=amax_send_sems.at[src],
                recv_sem=amax_recv_sems.at[src],
                device_id=(src,),
                device_id_type=pl.DeviceIdType.MESH,
            )
            rcv.wait_recv()

        amax = jnp.max(amax_ref[...])
        scale = amax / 448.0
        y = jnp.maximum(out_ref[...], 0.0)
        q = jnp.minimum(y / scale, 448.0)
        q = q.astype(jnp.float8_e4m3fn).astype(jnp.float32)
        out_ref[...] = q * scale

    out, _, _, _ = pl.pallas_call(
        body,
        out_shape=(
            jax.ShapeDtypeStruct((m_tot, n_per), jnp.float32),
            jax.ShapeDtypeStruct((N_DEV, PART_ROWS[0], k), jnp.float32),
            jax.ShapeDtypeStruct((N_DEV, PART_ROWS[1], k), jnp.float32),
            jax.ShapeDtypeStruct((N_DEV, PART_ROWS[2], k), jnp.float32),
        ),
        in_specs=[
            pl.BlockSpec(memory_space=pl.ANY),
            pl.BlockSpec(memory_space=pltpu.VMEM),
        ],
        out_specs=(
            pl.BlockSpec(memory_space=pltpu.VMEM),
            pl.BlockSpec(memory_space=pl.ANY),
            pl.BlockSpec(memory_space=pl.ANY),
            pl.BlockSpec(memory_space=pl.ANY),
        ),
        scratch_shapes=[
            pltpu.VMEM((4, PART_ROWS[0], k), jnp.float32),
            pltpu.VMEM((4, PART_ROWS[1], k), jnp.float32),
            pltpu.VMEM((4, PART_ROWS[2], k), jnp.float32),
            pltpu.VMEM((2, PART_ROWS[0], k), jnp.float32),
            pltpu.VMEM((N_DEV, 128), jnp.float32),
            pltpu.SemaphoreType.DMA((3, 7)),
            pltpu.SemaphoreType.DMA((3, 7)),
            pltpu.SemaphoreType.DMA((N_DEV,)),
            pltpu.SemaphoreType.DMA((N_DEV,)),
            pltpu.SemaphoreType.DMA((2,)),
        ],
        compiler_params=pltpu.CompilerParams(
            vmem_limit_bytes=52 * 1024 * 1024,
        ),
    )(x, w_mat)
    return out


# device time: 243684 ns/iter; 2.7589x vs baseline; 1.0179x over previous
import jax
import jax.numpy as jnp
from jax import lax
from jax.experimental import pallas as pl
from jax.experimental.pallas import tpu as pltpu

N_DEV = 8
PART_ROWS = (176, 168, 168)
PART_OFFS = (0, 176, 344)


def kernel(x, w_mat):
    m_per, k = x.shape
    _, n_per = w_mat.shape
    m_tot = N_DEV * m_per

    def body(x_ref, w_ref, out_ref, p0, p1, p2, land0, land1, land2,
             stage_ref, amax_ref, send_sems, recv_sems,
             amax_send_sems, amax_recv_sems, stage_sems):
        parts = (p0, p1, p2)
        landing = (land0, land1, land2)
        my = lax.axis_index("i")

        zc = my // 4
        p = my - 4 * zc
        yc = p // 2
        xc = jnp.bitwise_xor(p - 2 * yc, yc)

        def pos(x_, y_, z_):
            return 4 * z_ + 2 * y_ + jnp.bitwise_xor(x_, y_)

        nbr = (pos(1 - xc, yc, zc), pos(xc, 1 - yc, zc), pos(xc, yc, 1 - zc))

        barrier_sem = pltpu.get_barrier_semaphore()
        for e in range(3):
            pl.semaphore_signal(barrier_sem, inc=1, device_id=(nbr[e],),
                                device_id_type=pl.DeviceIdType.MESH)
        pl.semaphore_wait(barrier_sem, 3)

        def origin(j, s):
            c = [xc, yc, zc]
            for t in range(3):
                if (s >> t) & 1:
                    e = (j + t) % 3
                    c[e] = 1 - c[e]
            return pos(*c)

        def xslice(j):
            return x_ref.at[pl.ds(PART_OFFS[j], PART_ROWS[j])]

        running_amax = [jnp.float32(0.0)]

        def emit(j, yv, slot):
            running_amax[0] = jnp.maximum(running_amax[0], jnp.max(yv))
            out_ref[pl.ds(origin(j, slot) * m_per + PART_OFFS[j],
                          PART_ROWS[j]), :] = yv

        def dot(a):
            return jnp.dot(a, w_ref[...], preferred_element_type=jnp.float32,
                           precision=lax.Precision.HIGHEST)

        toggle = [0]
        queue = []

        def push(src, j, slot):
            if len(queue) == 2:
                finish_unit()
            s = toggle[0]
            toggle[0] ^= 1
            rows = PART_ROWS[j]
            cp = pltpu.make_async_copy(
                src, stage_ref.at[s, pl.ds(0, rows)], stage_sems.at[s]
            )
            cp.start()
            queue.append((cp, s, j, slot))

        def finish_unit():
            cp, s, j, slot = queue.pop(0)
            cp.wait()
            emit(j, dot(stage_ref[s, pl.ds(0, PART_ROWS[j])]), slot)

        def exchange(j, r, src, dst, sem_idx):
            return pltpu.make_async_remote_copy(
                src_ref=src,
                dst_ref=dst,
                send_sem=send_sems.at[j, sem_idx],
                recv_sem=recv_sems.at[j, sem_idx],
                device_id=(nbr[(j + r) % 3],),
                device_id_type=pl.DeviceIdType.MESH,
            )

        r0, r1a, r1b = [], [], []
        r2 = [[], [], [], []]
        for j in range(3):
            rd = exchange(j, 0, xslice(j), parts[j].at[1], sem_idx=0)
            rd.start()
            r0.append(rd)
            rd = exchange(j, 1, xslice(j), parts[j].at[2], sem_idx=1)
            rd.start()
            r1a.append(rd)
            rd = exchange(j, 2, xslice(j), landing[j].at[0], sem_idx=3)
            rd.start()
            r2[0].append(rd)
        for j in range(3):
            push(xslice(j), j, 0)

        for rd in r0:
            rd.wait()
        for j in range(3):
            rd = exchange(j, 1, parts[j].at[1], parts[j].at[3], sem_idx=2)
            rd.start()
            r1b.append(rd)
            rd = exchange(j, 2, parts[j].at[1], landing[j].at[1], sem_idx=4)
            rd.start()
            r2[1].append(rd)
        for j in range(3):
            push(parts[j].at[1], j, 1)

        for rd in r1a:
            rd.wait()
        for j in range(3):
            rd = exchange(j, 2, parts[j].at[2], landing[j].at[2], sem_idx=5)
            rd.start()
            r2[2].append(rd)
        for j in range(3):
            push(parts[j].at[2], j, 2)

        for rd in r1b:
            rd.wait()
        for j in range(3):
            rd = exchange(j, 2, parts[j].at[3], landing[j].at[3], sem_idx=6)
            rd.start()
            r2[3].append(rd)
        for j in range(3):
            push(parts[j].at[3], j, 3)

        for i in range(4):
            for rd in r2[i]:
                rd.wait()
            if i == 0:
                while queue:
                    finish_unit()
            for j in range(3):
                emit(j, dot(landing[j][i]), 4 + i)

        local_amax = jnp.maximum(running_amax[0], 0.0)
        amax_ref[pl.ds(my, 1), :] = jnp.full((1, 128), local_amax, jnp.float32)
        sends = []
        for j in range(1, N_DEV):
            tgt = lax.rem(my + j, N_DEV)
            a = pltpu.make_async_remote_copy(
                src_ref=amax_ref.at[pl.ds(my, 1)],
                dst_ref=amax_ref.at[pl.ds(my, 1)],
                send_sem=amax_send_sems.at[tgt],
                recv_sem=amax_recv_sems.at[my],
                device_id=(tgt,),
                device_id_type=pl.DeviceIdType.MESH,
            )
            a.start()
            sends.append(a)
        for a in sends:
            a.wait_send()
        for j in range(1, N_DEV):
            src = lax.rem(my + j, N_DEV)
            rcv = pltpu.make_async_remote_copy(
                src_ref=amax_ref.at[pl.ds(src, 1)],
                dst_ref=amax_ref.at[pl.ds(src, 1)],
                send_sem=amax_send_sems.at[src],
                recv_sem=amax_recv_sems.at[src],
                device_id=(src,),
                device_id_type=pl.DeviceIdType.MESH,
            )
            rcv.wait_recv()

        amax = jnp.max(amax_ref[...])
        scale = amax / 448.0
        y = jnp.maximum(out_ref[...], 0.0)
        q = jnp.minimum(y / scale, 448.0)
        q = q.astype(jnp.float8_e4m3fn).astype(jnp.float32)
        out_ref[...] = q * scale

    out, _, _, _ = pl.pallas_call(
        body,
        out_shape=(
            jax.ShapeDtypeStruct((m_tot, n_per), jnp.float32),
            jax.ShapeDtypeStruct((N_DEV, PART_ROWS[0], k), jnp.float32),
            jax.ShapeDtypeStruct((N_DEV, PART_ROWS[1], k), jnp.float32),
            jax.ShapeDtypeStruct((N_DEV, PART_ROWS[2], k), jnp.float32),
        ),
        in_specs=[
            pl.BlockSpec(memory_space=pl.ANY),
            pl.BlockSpec(memory_space=pltpu.VMEM),
        ],
        out_specs=(
            pl.BlockSpec(memory_space=pltpu.VMEM),
            pl.BlockSpec(memory_space=pl.ANY),
            pl.BlockSpec(memory_space=pl.ANY),
            pl.BlockSpec(memory_space=pl.ANY),
        ),
        scratch_shapes=[
            pltpu.VMEM((4, PART_ROWS[0], k), jnp.float32),
            pltpu.VMEM((4, PART_ROWS[1], k), jnp.float32),
            pltpu.VMEM((4, PART_ROWS[2], k), jnp.float32),
            pltpu.VMEM((2, PART_ROWS[0], k), jnp.float32),
            pltpu.VMEM((N_DEV, 128), jnp.float32),
            pltpu.SemaphoreType.DMA((3, 7)),
            pltpu.SemaphoreType.DMA((3, 7)),
            pltpu.SemaphoreType.DMA((N_DEV,)),
            pltpu.SemaphoreType.DMA((N_DEV,)),
            pltpu.SemaphoreType.DMA((2,)),
        ],
        compiler_params=pltpu.CompilerParams(
            vmem_limit_bytes=52 * 1024 * 1024,
            collective_id=0,
        ),
    )(x, w_mat)
    return out
